# baseline (speedup 1.0000x reference)
import jax
import jax.numpy as jnp
from jax import lax
from jax.experimental import pallas as pl
from jax.experimental.pallas import tpu as pltpu

_BLOCK_B = 128
_SLAB_K = 8


def _rotl(x, d):
    return lax.shift_left(x, jnp.uint32(d)) | lax.shift_right_logical(
        x, jnp.uint32(32 - d))


def _threefry_rounds(x0, x1, rots):
    for r in rots:
        x0 = x0 + x1
        x1 = _rotl(x1, r)
        x1 = x1 ^ x0
    return x0, x1


def _gumbel_key42(cnt):
    ks0 = jnp.uint32(0)
    ks1 = jnp.uint32(42)
    ks2 = ks0 ^ ks1 ^ jnp.uint32(0x1BD11BDA)
    rot0 = (13, 15, 26, 6)
    rot1 = (17, 29, 16, 24)
    x0 = jnp.zeros_like(cnt) + ks0
    x1 = cnt + ks1
    x0, x1 = _threefry_rounds(x0, x1, rot0)
    x0 = x0 + ks1
    x1 = x1 + ks2 + jnp.uint32(1)
    x0, x1 = _threefry_rounds(x0, x1, rot1)
    x0 = x0 + ks2
    x1 = x1 + ks0 + jnp.uint32(2)
    x0, x1 = _threefry_rounds(x0, x1, rot0)
    x0 = x0 + ks0
    x1 = x1 + ks1 + jnp.uint32(3)
    x0, x1 = _threefry_rounds(x0, x1, rot1)
    x0 = x0 + ks1
    x1 = x1 + ks2 + jnp.uint32(4)
    x0, x1 = _threefry_rounds(x0, x1, rot0)
    x0 = x0 + ks2
    x1 = x1 + ks0 + jnp.uint32(5)
    bits = x0 ^ x1
    fb = lax.shift_right_logical(bits, jnp.uint32(9)) | jnp.uint32(0x3F800000)
    f = lax.bitcast_convert_type(fb, jnp.float32) - jnp.float32(1.0)
    tiny = jnp.float32(1.1754944e-38)
    u = jnp.maximum(tiny, f * (jnp.float32(1.0) - tiny) + tiny)
    return -jnp.log(-jnp.log(u))


def _gibbs_block(mus_ref, sigmas_ref, xs_ref, z_ref):
    x = xs_ref[...][:, None, :]
    BB, K = mus_ref.shape[0], mus_ref.shape[1]
    parts = []
    for k0 in range(0, K, _SLAB_K):
        mus = mus_ref[:, k0:k0 + _SLAB_K, :]
        sigmas = sigmas_ref[:, k0:k0 + _SLAB_K, :]
        log_prob = (-0.5 * ((x - mus) / sigmas) ** 2
                    - jnp.log(sigmas)
                    - 0.5 * jnp.log(2.0 * jnp.pi))
        parts.append(log_prob.sum(axis=-1))
    logits = jnp.concatenate(parts, axis=-1)
    r0 = (pl.program_id(0) * BB).astype(jnp.uint32)
    row = lax.broadcasted_iota(jnp.uint32, (BB, K), 0) + r0
    col = lax.broadcasted_iota(jnp.uint32, (BB, K), 1)
    g = _gumbel_key42(row * jnp.uint32(K) + col)
    z = jnp.argmax(logits + g, axis=-1)
    z_ref[0, 0, :] = z.astype(jnp.int32)


def kernel(mus, sigmas, xs):
    B, K, D = mus.shape
    bb = _BLOCK_B
    nb = B // bb
    z_blocks = pl.pallas_call(
        _gibbs_block,
        grid=(nb,),
        in_specs=[
            pl.BlockSpec((bb, K, D), lambda i: (i, 0, 0)),
            pl.BlockSpec((bb, K, D), lambda i: (i, 0, 0)),
            pl.BlockSpec((bb, D), lambda i: (i, 0)),
        ],
        out_specs=pl.BlockSpec((1, 1, bb), lambda i: (i, 0, 0)),
        out_shape=jax.ShapeDtypeStruct((nb, 1, bb), jnp.int32),
        compiler_params=pltpu.CompilerParams(
            dimension_semantics=("parallel",),
        ),
    )(mus, sigmas, xs)
    return (z_blocks.reshape(B), xs)


# speedup vs baseline: 1.0170x; 1.0170x over previous
import jax
import jax.numpy as jnp
from jax import lax
from jax.experimental import pallas as pl
from jax.experimental.pallas import tpu as pltpu

_BLOCK_B = 128
_SLAB_K = 8


def _rotl(x, d):
    return lax.shift_left(x, jnp.uint32(d)) | lax.shift_right_logical(
        x, jnp.uint32(32 - d))


def _threefry_rounds(x0, x1, rots):
    for r in rots:
        x0 = x0 + x1
        x1 = _rotl(x1, r)
        x1 = x1 ^ x0
    return x0, x1


def _gumbel_key42(cnt):
    ks0 = jnp.uint32(0)
    ks1 = jnp.uint32(42)
    ks2 = ks0 ^ ks1 ^ jnp.uint32(0x1BD11BDA)
    rot0 = (13, 15, 26, 6)
    rot1 = (17, 29, 16, 24)
    x0 = jnp.zeros_like(cnt) + ks0
    x1 = cnt + ks1
    x0, x1 = _threefry_rounds(x0, x1, rot0)
    x0 = x0 + ks1
    x1 = x1 + ks2 + jnp.uint32(1)
    x0, x1 = _threefry_rounds(x0, x1, rot1)
    x0 = x0 + ks2
    x1 = x1 + ks0 + jnp.uint32(2)
    x0, x1 = _threefry_rounds(x0, x1, rot0)
    x0 = x0 + ks0
    x1 = x1 + ks1 + jnp.uint32(3)
    x0, x1 = _threefry_rounds(x0, x1, rot1)
    x0 = x0 + ks1
    x1 = x1 + ks2 + jnp.uint32(4)
    x0, x1 = _threefry_rounds(x0, x1, rot0)
    x0 = x0 + ks2
    x1 = x1 + ks0 + jnp.uint32(5)
    bits = x0 ^ x1
    fb = lax.shift_right_logical(bits, jnp.uint32(9)) | jnp.uint32(0x3F800000)
    f = lax.bitcast_convert_type(fb, jnp.float32) - jnp.float32(1.0)
    tiny = jnp.float32(1.1754944e-38)
    u = jnp.maximum(tiny, f * (jnp.float32(1.0) - tiny) + tiny)
    return -jnp.log(-jnp.log(u))


def _logits_half(mus_ref, sigmas_ref, x):
    BB, K = mus_ref.shape[0], mus_ref.shape[1]
    parts = []
    for k0 in range(0, K, _SLAB_K):
        mus = mus_ref[:, k0:k0 + _SLAB_K, :]
        sigmas = sigmas_ref[:, k0:k0 + _SLAB_K, :]
        log_prob = (-0.5 * ((x - mus) / sigmas) ** 2
                    - jnp.log(sigmas)
                    - 0.5 * jnp.log(2.0 * jnp.pi))
        parts.append(log_prob.sum(axis=-1))
    return jnp.concatenate(parts, axis=-1)


def _gibbs_block(mus_lo, sigmas_lo, xs_lo, mus_hi, sigmas_hi, xs_hi, z_ref):
    BB, K = mus_lo.shape[0], mus_lo.shape[1]
    half = jnp.uint32(2048 * K)
    r0 = (pl.program_id(0) * BB).astype(jnp.uint32)
    row = lax.broadcasted_iota(jnp.uint32, (BB, K), 0) + r0
    col = lax.broadcasted_iota(jnp.uint32, (BB, K), 1)
    cnt = row * jnp.uint32(K) + col
    logits_lo = _logits_half(mus_lo, sigmas_lo, xs_lo[...][:, None, :])
    z_lo = jnp.argmax(logits_lo + _gumbel_key42(cnt), axis=-1)
    logits_hi = _logits_half(mus_hi, sigmas_hi, xs_hi[...][:, None, :])
    z_hi = jnp.argmax(logits_hi + _gumbel_key42(cnt + half), axis=-1)
    z_ref[0, 0, :] = z_lo.astype(jnp.int32)
    z_ref[0, 1, :] = z_hi.astype(jnp.int32)


def kernel(mus, sigmas, xs):
    B, K, D = mus.shape
    bb = _BLOCK_B
    nb = (B // 2) // bb
    off = nb
    z_blocks = pl.pallas_call(
        _gibbs_block,
        grid=(nb,),
        in_specs=[
            pl.BlockSpec((bb, K, D), lambda i: (i, 0, 0)),
            pl.BlockSpec((bb, K, D), lambda i: (i, 0, 0)),
            pl.BlockSpec((bb, D), lambda i: (i, 0)),
            pl.BlockSpec((bb, K, D), lambda i: (i + off, 0, 0)),
            pl.BlockSpec((bb, K, D), lambda i: (i + off, 0, 0)),
            pl.BlockSpec((bb, D), lambda i: (i + off, 0)),
        ],
        out_specs=pl.BlockSpec((1, 2, bb), lambda i: (i, 0, 0)),
        out_shape=jax.ShapeDtypeStruct((nb, 2, bb), jnp.int32),
        compiler_params=pltpu.CompilerParams(
            dimension_semantics=("parallel",),
        ),
    )(mus, sigmas, xs, mus, sigmas, xs)
    z = jnp.concatenate([z_blocks[:, 0, :].reshape(B // 2),
                         z_blocks[:, 1, :].reshape(B // 2)])
    return (z, xs)


# speedup vs baseline: 1.0599x; 1.0421x over previous
import jax
import jax.numpy as jnp
from jax import lax
from jax.experimental import pallas as pl
from jax.experimental.pallas import tpu as pltpu

_BLOCK_B = 256
_SLAB_K = 8


def _rotl(x, d):
    return lax.shift_left(x, jnp.uint32(d)) | lax.shift_right_logical(
        x, jnp.uint32(32 - d))


def _threefry_rounds(x0, x1, rots):
    for r in rots:
        x0 = x0 + x1
        x1 = _rotl(x1, r)
        x1 = x1 ^ x0
    return x0, x1


def _gumbel_key42(cnt):
    ks0 = jnp.uint32(0)
    ks1 = jnp.uint32(42)
    ks2 = ks0 ^ ks1 ^ jnp.uint32(0x1BD11BDA)
    rot0 = (13, 15, 26, 6)
    rot1 = (17, 29, 16, 24)
    x0 = jnp.zeros_like(cnt) + ks0
    x1 = cnt + ks1
    x0, x1 = _threefry_rounds(x0, x1, rot0)
    x0 = x0 + ks1
    x1 = x1 + ks2 + jnp.uint32(1)
    x0, x1 = _threefry_rounds(x0, x1, rot1)
    x0 = x0 + ks2
    x1 = x1 + ks0 + jnp.uint32(2)
    x0, x1 = _threefry_rounds(x0, x1, rot0)
    x0 = x0 + ks0
    x1 = x1 + ks1 + jnp.uint32(3)
    x0, x1 = _threefry_rounds(x0, x1, rot1)
    x0 = x0 + ks1
    x1 = x1 + ks2 + jnp.uint32(4)
    x0, x1 = _threefry_rounds(x0, x1, rot0)
    x0 = x0 + ks2
    x1 = x1 + ks0 + jnp.uint32(5)
    bits = x0 ^ x1
    fb = lax.shift_right_logical(bits, jnp.uint32(9)) | jnp.uint32(0x3F800000)
    f = lax.bitcast_convert_type(fb, jnp.float32) - jnp.float32(1.0)
    tiny = jnp.float32(1.1754944e-38)
    u = jnp.maximum(tiny, f * (jnp.float32(1.0) - tiny) + tiny)
    return -jnp.log(-jnp.log(u))


def _gibbs_block(mus_ref, sigmas_ref, xs_ref, z_ref):
    x = xs_ref[...][:, None, :]
    BB, K = mus_ref.shape[0], mus_ref.shape[1]
    parts = []
    for k0 in range(0, K, _SLAB_K):
        mus = mus_ref[:, k0:k0 + _SLAB_K, :]
        sigmas = sigmas_ref[:, k0:k0 + _SLAB_K, :]
        log_prob = (-0.5 * ((x - mus) / sigmas) ** 2
                    - jnp.log(sigmas)
                    - 0.5 * jnp.log(2.0 * jnp.pi))
        parts.append(log_prob.sum(axis=-1))
    logits = jnp.concatenate(parts, axis=-1)
    r0 = (pl.program_id(0) * BB).astype(jnp.uint32)
    row = lax.broadcasted_iota(jnp.uint32, (BB, K), 0) + r0
    col = lax.broadcasted_iota(jnp.uint32, (BB, K), 1)
    g = _gumbel_key42(row * jnp.uint32(K) + col)
    z = jnp.argmax(logits + g, axis=-1)
    z_ref[0, 0, :] = z.astype(jnp.int32)


def kernel(mus, sigmas, xs):
    B, K, D = mus.shape
    bb = _BLOCK_B
    nb = B // bb
    z_blocks = pl.pallas_call(
        _gibbs_block,
        grid=(nb,),
        in_specs=[
            pl.BlockSpec((bb, K, D), lambda i: (i, 0, 0)),
            pl.BlockSpec((bb, K, D), lambda i: (i, 0, 0)),
            pl.BlockSpec((bb, D), lambda i: (i, 0)),
        ],
        out_specs=pl.BlockSpec((1, 1, bb), lambda i: (i, 0, 0)),
        out_shape=jax.ShapeDtypeStruct((nb, 1, bb), jnp.int32),
        compiler_params=pltpu.CompilerParams(
            dimension_semantics=("parallel",),
        ),
    )(mus, sigmas, xs)
    return (z_blocks.reshape(B), xs)
